# baseline (speedup 1.0000x reference)
import functools

import jax
import jax.numpy as jnp
from jax import lax
from jax.experimental import pallas as pl
from jax.experimental.pallas import tpu as pltpu
from jax.experimental.pallas import tpu_sc as plsc

_BATCH = 16384
_NSEG = 26
_SEG = 128


def kernel(pooled_embs):
    x = pooled_embs.reshape(_BATCH, _NSEG, _SEG)

    info = plsc.get_sparse_core_info()
    num_workers = info.num_cores * info.num_subcores
    rows_per = _BATCH // num_workers

    mesh = plsc.VectorSubcoreMesh(core_axis_name="c", subcore_axis_name="s")

    @functools.partial(
        pl.kernel,
        mesh=mesh,
        out_type=jax.ShapeDtypeStruct((_BATCH, _NSEG, _SEG), jnp.float32),
    )
    def permute_sc(in_hbm, out_hbm):
        wid = lax.axis_index("s") * info.num_cores + lax.axis_index("c")
        base = wid * rows_per
        for j in range(_NSEG):
            pltpu.sync_copy(
                in_hbm.at[pl.ds(base, rows_per), _NSEG - 1 - j],
                out_hbm.at[pl.ds(base, rows_per), j],
            )

    return permute_sc(x).reshape(_BATCH, _NSEG * _SEG)


# speedup vs baseline: 12.5215x; 12.5215x over previous
import functools

import numpy as np
import jax
import jax.numpy as jnp
from jax import lax
from jax.experimental import pallas as pl
from jax.experimental.pallas import tpu as pltpu
from jax.experimental.pallas import tpu_sc as plsc

_BATCH = 16384
_NSEG = 26
_SEG = 128
_ROWS = _BATCH * _NSEG

_CHUNK = 128
_NBUF = 4


def _make_index_array(num_workers, nch):
    i = np.arange(_ROWS, dtype=np.int64)
    idx = (i // _NSEG) * _NSEG + (_NSEG - 1 - i % _NSEG)
    return jnp.asarray(idx.reshape(num_workers, nch, _CHUNK).astype(np.int32))


def kernel(pooled_embs):
    x = pooled_embs.reshape(_ROWS, _SEG)

    info = plsc.get_sparse_core_info()
    num_workers = info.num_cores * info.num_subcores
    rows_per = _ROWS // num_workers
    nch = rows_per // _CHUNK

    idx_arr = _make_index_array(num_workers, nch)

    mesh = plsc.VectorSubcoreMesh(core_axis_name="c", subcore_axis_name="s")

    @functools.partial(
        pl.kernel,
        mesh=mesh,
        out_type=jax.ShapeDtypeStruct((_ROWS, _SEG), jnp.float32),
        scratch_types=[
            pltpu.VMEM((nch, _CHUNK), jnp.int32),
            pltpu.VMEM((_NBUF, _CHUNK, _SEG), jnp.float32),
        ] + [pltpu.SemaphoreType.DMA] * _NBUF,
    )
    def permute_sc(in_hbm, idx_hbm, out_hbm, idx_v, rows_v, *sems):
        wid = lax.axis_index("s") * info.num_cores + lax.axis_index("c")
        base = wid * rows_per

        pltpu.sync_copy(idx_hbm.at[wid], idx_v)

        for b in range(_NBUF):
            pltpu.async_copy(in_hbm.at[idx_v.at[b]], rows_v.at[b], sems[b])

        @pl.loop(0, nch, step=_NBUF)
        def _ring(g):
            for b in range(_NBUF):
                c = g + b
                pltpu.make_async_copy(
                    in_hbm.at[idx_v.at[c]], rows_v.at[b], sems[b]
                ).wait()
                pltpu.sync_copy(
                    rows_v.at[b], out_hbm.at[pl.ds(base + c * _CHUNK, _CHUNK)]
                )
                nxt = c + _NBUF

                @pl.when(nxt < nch)
                def _():
                    pltpu.async_copy(
                        in_hbm.at[idx_v.at[nxt]], rows_v.at[b], sems[b]
                    )

    return permute_sc(x, idx_arr).reshape(_BATCH, _NSEG * _SEG)
